# speedup vs baseline: 4.2695x; 4.2695x over previous
import functools

import jax
import jax.numpy as jnp
from jax import lax
from jax.experimental import pallas as pl
from jax.experimental.pallas import tpu as pltpu
from jax.experimental.pallas import tpu_sc as plsc

_info = plsc.get_sparse_core_info()
_NC = _info.num_cores
_NS = _info.num_subcores
_LANES = _info.num_lanes
_NW = _NC * _NS


def _emb_body(seq_per_w, seq_len, embed_dim,
              ids_hbm, tok_hbm, pos_hbm, out_hbm,
              pos_v, idx_v, rows_v, gsem):
    wid = lax.axis_index("s") * _NC + lax.axis_index("c")

    pltpu.sync_copy(pos_hbm.at[pl.ds(0, seq_len)], pos_v)

    n_chunks = embed_dim // _LANES

    def seq_body(i, carry):
        gseq = wid * seq_per_w + i
        base = gseq * seq_len
        pltpu.sync_copy(ids_hbm.at[pl.ds(base, seq_len)], idx_v)
        cp1 = pltpu.async_copy(tok_hbm.at[idx_v.at[pl.ds(0, 128)]],
                               rows_v.at[pl.ds(0, 128)], gsem)
        cp2 = pltpu.async_copy(tok_hbm.at[idx_v.at[pl.ds(128, seq_len - 128)]],
                               rows_v.at[pl.ds(128, seq_len - 128)], gsem)
        cp1.wait()
        cp2.wait()

        def add_row(r, c2):
            for c in range(n_chunks):
                plsc.addupdate(rows_v.at[r, pl.ds(c * _LANES, _LANES)],
                               pos_v[r, pl.ds(c * _LANES, _LANES)])
            return c2

        lax.fori_loop(0, seq_len, add_row, 0)

        pltpu.sync_copy(rows_v, out_hbm.at[pl.ds(base, seq_len)])
        return carry

    lax.fori_loop(0, seq_per_w, seq_body, 0)


def kernel(input_ids, token_table, pos_table):
    batch, seq_len = input_ids.shape
    vocab, embed_dim = token_table.shape
    seq_per_w = batch // _NW

    ids_flat = input_ids.reshape(-1).astype(jnp.int32)

    mesh = plsc.VectorSubcoreMesh(core_axis_name="c", subcore_axis_name="s")
    body = functools.partial(_emb_body, seq_per_w, seq_len, embed_dim)
    out = pl.kernel(
        body,
        out_type=jax.ShapeDtypeStruct((batch * seq_len, embed_dim),
                                      jnp.float32),
        mesh=mesh,
        scratch_types=[
            pltpu.VMEM((seq_len, embed_dim), jnp.float32),
            pltpu.VMEM((seq_len,), jnp.int32),
            pltpu.VMEM((seq_len, embed_dim), jnp.float32),
            pltpu.SemaphoreType.DMA,
        ],
    )(ids_flat, token_table, pos_table)
    return out.reshape(batch, seq_len, embed_dim)


# speedup vs baseline: 7.5045x; 1.7577x over previous
import functools

import jax
import jax.numpy as jnp
from jax import lax
from jax.experimental import pallas as pl
from jax.experimental.pallas import tpu as pltpu
from jax.experimental.pallas import tpu_sc as plsc

_info = plsc.get_sparse_core_info()
_NC = _info.num_cores
_NS = _info.num_subcores
_LANES = _info.num_lanes
_NW = _NC * _NS


def _emb_body(seq_per_w, seq_len, embed_dim,
              ids_hbm, tok_hbm, pos_hbm, out_hbm,
              pos_v, idx_v, rows0, rows1,
              gsem0, gsem1, ssem0, ssem1):
    wid = lax.axis_index("s") * _NC + lax.axis_index("c")
    wbase = wid * seq_per_w

    pltpu.sync_copy(pos_hbm.at[pl.ds(0, seq_len)], pos_v)
    pltpu.sync_copy(ids_hbm.at[pl.ds(wbase * seq_len, seq_per_w * seq_len)],
                    idx_v)

    split = min(128, seq_len)
    rest = seq_len - split

    def g_descs(i, buf, sem):
        off = i * seq_len
        ds = [pltpu.make_async_copy(tok_hbm.at[idx_v.at[pl.ds(off, split)]],
                                    buf.at[pl.ds(0, split)], sem)]
        if rest:
            ds.append(pltpu.make_async_copy(
                tok_hbm.at[idx_v.at[pl.ds(off + split, rest)]],
                buf.at[pl.ds(split, rest)], sem))
        return ds

    def s_desc(i, buf, sem):
        return pltpu.make_async_copy(
            buf, out_hbm.at[pl.ds((wbase + i) * seq_len, seq_len)], sem)

    def add_pos(buf):
        def add_row(r, carry):
            for c in range(embed_dim // _LANES):
                plsc.addupdate(buf.at[r, pl.ds(c * _LANES, _LANES)],
                               pos_v[r, pl.ds(c * _LANES, _LANES)])
            return carry
        lax.fori_loop(0, seq_len, add_row, 0)

    for d in g_descs(0, rows0, gsem0):
        d.start()

    n_pairs = seq_per_w // 2

    def pair(p, carry):
        i0 = 2 * p
        i1 = i0 + 1

        @pl.when(p > 0)
        def _():
            s_desc(i0 - 1, rows1, ssem1).wait()
        for d in g_descs(i1, rows1, gsem1):
            d.start()
        for d in g_descs(i0, rows0, gsem0):
            d.wait()
        add_pos(rows0)
        s_desc(i0, rows0, ssem0).start()

        s_desc(i0, rows0, ssem0).wait()
        @pl.when(p < n_pairs - 1)
        def _():
            for d in g_descs(i1 + 1, rows0, gsem0):
                d.start()
        for d in g_descs(i1, rows1, gsem1):
            d.wait()
        add_pos(rows1)
        s_desc(i1, rows1, ssem1).start()
        return carry

    lax.fori_loop(0, n_pairs, pair, 0)
    s_desc(seq_per_w - 1, rows1, ssem1).wait()


def kernel(input_ids, token_table, pos_table):
    batch, seq_len = input_ids.shape
    vocab, embed_dim = token_table.shape
    seq_per_w = batch // _NW

    ids_flat = input_ids.reshape(-1).astype(jnp.int32)

    mesh = plsc.VectorSubcoreMesh(core_axis_name="c", subcore_axis_name="s")
    body = functools.partial(_emb_body, seq_per_w, seq_len, embed_dim)
    out = pl.kernel(
        body,
        out_type=jax.ShapeDtypeStruct((batch * seq_len, embed_dim),
                                      jnp.float32),
        mesh=mesh,
        scratch_types=[
            pltpu.VMEM((seq_len, embed_dim), jnp.float32),
            pltpu.VMEM((seq_per_w * seq_len,), jnp.int32),
            pltpu.VMEM((seq_len, embed_dim), jnp.float32),
            pltpu.VMEM((seq_len, embed_dim), jnp.float32),
            pltpu.SemaphoreType.DMA,
            pltpu.SemaphoreType.DMA,
            pltpu.SemaphoreType.DMA,
            pltpu.SemaphoreType.DMA,
        ],
    )(ids_flat, token_table, pos_table)
    return out.reshape(batch, seq_len, embed_dim)


# speedup vs baseline: 9.0246x; 1.2026x over previous
import functools

import jax
import jax.numpy as jnp
from jax import lax
from jax.experimental import pallas as pl
from jax.experimental.pallas import tpu as pltpu
from jax.experimental.pallas import tpu_sc as plsc

_info = plsc.get_sparse_core_info()
_NC = _info.num_cores
_NS = _info.num_subcores
_LANES = _info.num_lanes
_NW = _NC * _NS


def _emb_body(seq_per_w, seq_len, embed_dim,
              ids_hbm, tok_hbm, pos_hbm, out_hbm,
              pos_v, idx_v, rows0, rows1, rows2,
              gsem0, gsem1, gsem2, ssem0, ssem1, ssem2):
    wid = lax.axis_index("s") * _NC + lax.axis_index("c")
    wbase = wid * seq_per_w

    pltpu.sync_copy(pos_hbm.at[pl.ds(0, seq_len)], pos_v)
    pltpu.sync_copy(ids_hbm.at[pl.ds(wbase * seq_len, seq_per_w * seq_len)],
                    idx_v)

    split = min(128, seq_len)
    rest = seq_len - split

    def g_descs(i, buf, sem):
        off = i * seq_len
        ds = [pltpu.make_async_copy(tok_hbm.at[idx_v.at[pl.ds(off, split)]],
                                    buf.at[pl.ds(0, split)], sem)]
        if rest:
            ds.append(pltpu.make_async_copy(
                tok_hbm.at[idx_v.at[pl.ds(off + split, rest)]],
                buf.at[pl.ds(split, rest)], sem))
        return ds

    def s_desc(i, buf, sem):
        return pltpu.make_async_copy(
            buf, out_hbm.at[pl.ds((wbase + i) * seq_len, seq_len)], sem)

    def add_pos(buf):
        def add_row(r, carry):
            for c in range(embed_dim // _LANES):
                plsc.addupdate(buf.at[r, pl.ds(c * _LANES, _LANES)],
                               pos_v[r, pl.ds(c * _LANES, _LANES)])
            return carry
        lax.fori_loop(0, seq_len, add_row, 0)

    bufs = ((rows0, gsem0, ssem0), (rows1, gsem1, ssem1),
            (rows2, gsem2, ssem2))
    n_seq = seq_per_w

    def step(i, cur, nxt):
        buf, gsem, ssem = cur
        nbuf, ngsem, nssem = nxt
        @pl.when(i >= 2)
        def _():
            s_desc(i - 2, nbuf, nssem).wait()
        @pl.when(i + 1 < n_seq)
        def _():
            for d in g_descs(i + 1, nbuf, ngsem):
                d.start()
        for d in g_descs(i, buf, gsem):
            d.wait()
        add_pos(buf)
        s_desc(i, buf, ssem).start()

    for d in g_descs(0, rows0, gsem0):
        d.start()

    n_triples = n_seq // 3

    def triple(t, carry):
        for k in range(3):
            step(3 * t + k, bufs[k], bufs[(k + 1) % 3])
        return carry

    lax.fori_loop(0, n_triples, triple, 0)
    for i in range(3 * n_triples, n_seq):
        step(i, bufs[i % 3], bufs[(i + 1) % 3])
    s_desc(n_seq - 2, bufs[(n_seq - 2) % 3][0], bufs[(n_seq - 2) % 3][2]).wait()
    s_desc(n_seq - 1, bufs[(n_seq - 1) % 3][0], bufs[(n_seq - 1) % 3][2]).wait()


def kernel(input_ids, token_table, pos_table):
    batch, seq_len = input_ids.shape
    vocab, embed_dim = token_table.shape
    seq_per_w = batch // _NW

    ids_flat = input_ids.reshape(-1).astype(jnp.int32)

    mesh = plsc.VectorSubcoreMesh(core_axis_name="c", subcore_axis_name="s")
    body = functools.partial(_emb_body, seq_per_w, seq_len, embed_dim)
    out = pl.kernel(
        body,
        out_type=jax.ShapeDtypeStruct((batch * seq_len, embed_dim),
                                      jnp.float32),
        mesh=mesh,
        scratch_types=[
            pltpu.VMEM((seq_len, embed_dim), jnp.float32),
            pltpu.VMEM((seq_per_w * seq_len,), jnp.int32),
            pltpu.VMEM((seq_len, embed_dim), jnp.float32),
            pltpu.VMEM((seq_len, embed_dim), jnp.float32),
            pltpu.VMEM((seq_len, embed_dim), jnp.float32),
            pltpu.SemaphoreType.DMA,
            pltpu.SemaphoreType.DMA,
            pltpu.SemaphoreType.DMA,
            pltpu.SemaphoreType.DMA,
            pltpu.SemaphoreType.DMA,
            pltpu.SemaphoreType.DMA,
        ],
    )(ids_flat, token_table, pos_table)
    return out.reshape(batch, seq_len, embed_dim)
